# baseline (speedup 1.0000x reference)
import jax
import jax.numpy as jnp
from jax.experimental import pallas as pl


def _wsum_kernel(x_ref, v_ref, o_ref):
    xb = x_ref[...].astype(jnp.bfloat16).astype(jnp.float32)
    vb = v_ref[...].astype(jnp.bfloat16).astype(jnp.float32)
    o_ref[...] = jnp.sum(xb * vb, axis=(0, 1), keepdims=True)


def kernel(x, values):
    out = pl.pallas_call(
        _wsum_kernel,
        out_shape=jax.ShapeDtypeStruct((1, 1), jnp.float32),
    )(x, values.reshape(2, 1))
    return out[0, 0]
